# baseline (speedup 1.0000x reference)
import functools

import jax
import jax.numpy as jnp
from jax import lax
from jax.experimental import pallas as pl
from jax.experimental.pallas import tpu as pltpu
from jax.experimental.pallas import tpu_sc as plsc

NC = 2
NS = 16
NW = NC * NS
L = 16

B = 16384
D = 32
BPW = B // NW
CHUNK = 128
NCHUNK = BPW // CHUNK
GROUPS = BPW // L


def _mf_body(uid_hbm, iid_hbm, uet_hbm, ubt_hbm, iet_hbm, ibt_hbm,
             out_hbm, uid_v, iid_v, urt, irt, ub, ib, out_v, sem):
    wid = lax.axis_index("s") * NC + lax.axis_index("c")
    base = wid * BPW

    for c in range(NCHUNK):
        pltpu.sync_copy(uid_hbm.at[pl.ds(base + c * CHUNK, CHUNK)], uid_v.at[c])
        pltpu.sync_copy(iid_hbm.at[pl.ds(base + c * CHUNK, CHUNK)], iid_v.at[c])

    bias_copies = []
    for c in range(NCHUNK):
        sl = pl.ds(c * CHUNK, CHUNK)
        bias_copies.append(
            pltpu.async_copy(ubt_hbm.at[0].at[uid_v.at[c]], ub.at[sl], sem))
        bias_copies.append(
            pltpu.async_copy(ibt_hbm.at[0].at[iid_v.at[c]], ib.at[sl], sem))

    def issue(d, carry):
        for c in range(NCHUNK):
            sl = pl.ds(c * CHUNK, CHUNK)
            pltpu.async_copy(uet_hbm.at[d].at[uid_v.at[c]], urt.at[d, sl], sem)
            pltpu.async_copy(iet_hbm.at[d].at[iid_v.at[c]], irt.at[d, sl], sem)
        return carry

    lax.fori_loop(0, D, issue, 0)

    for cp in bias_copies:
        cp.wait()
    pltpu.make_async_copy(uet_hbm.at[:, pl.ds(0, BPW)], urt, sem).wait()
    pltpu.make_async_copy(iet_hbm.at[:, pl.ds(0, BPW)], irt, sem).wait()

    def group(g, carry):
        r0 = pl.multiple_of(g * L, L)
        sl = pl.ds(r0, L)
        acc = ub[sl] + ib[sl]
        for d in range(D):
            acc = acc + urt[d, sl] * irt[d, sl]
        out_v[sl] = acc
        return carry

    lax.fori_loop(0, GROUPS, group, 0)
    pltpu.sync_copy(out_v, out_hbm.at[pl.ds(base, BPW)])


_mf_kernel = functools.partial(
    pl.kernel,
    out_type=jax.ShapeDtypeStruct((B,), jnp.float32),
    mesh=plsc.VectorSubcoreMesh(
        core_axis_name="c", subcore_axis_name="s",
        num_cores=NC, num_subcores=NS),
    scratch_types=[
        pltpu.VMEM((NCHUNK, CHUNK), jnp.int32),
        pltpu.VMEM((NCHUNK, CHUNK), jnp.int32),
        pltpu.VMEM((D, BPW), jnp.float32),
        pltpu.VMEM((D, BPW), jnp.float32),
        pltpu.VMEM((BPW,), jnp.float32),
        pltpu.VMEM((BPW,), jnp.float32),
        pltpu.VMEM((BPW,), jnp.float32),
        pltpu.SemaphoreType.DMA,
    ],
    compiler_params=pltpu.CompilerParams(needs_layout_passes=False,
                                         use_tc_tiling_on_sc=False),
)(_mf_body)


@jax.jit
def kernel(user_id, item_id, user_embedding, user_bias, item_embedding,
           item_bias):
    uid = user_id.astype(jnp.int32)
    iid = item_id.astype(jnp.int32)
    return _mf_kernel(uid, iid, user_embedding.T, user_bias.T,
                      item_embedding.T, item_bias.T)
